# baseline (speedup 1.0000x reference)
import functools

import jax
import jax.numpy as jnp
from jax import lax
from jax.experimental import pallas as pl
from jax.experimental.pallas import tpu as pltpu
from jax.experimental.pallas import tpu_sc as plsc

_NC = 2
_NS = 16
_NW = _NC * _NS

_HIST = 50
_D = 64
_PAIR = 2
_G = _PAIR * _HIST
_NBUF = 4


def _fmt_body(tt_hbm, out_hbm, buf0, buf1, buft, row_v, sem0, sem1):
    d, v = tt_hbm.shape
    ncol = v // 128
    rem = v - 128 * ncol
    base, extra = ncol // _NW, ncol % _NW
    wid = lax.axis_index("s") * _NC + lax.axis_index("c")
    start = base * wid + jnp.minimum(wid, extra)
    count = base + (wid < extra).astype(jnp.int32)

    def _transpose(buf, nl, row_ref):
        def _tb(l, _):
            half = (l % 2) * _D
            for c in range(4):
                rows = 16 * c + lax.iota(jnp.int32, 16)
                cols = jnp.full((16,), l, jnp.int32)
                g = plsc.load_gather(buf, [rows, cols])
                row_ref[l // 2, pl.ds(half + 16 * c, 16)] = g
            return 0

        lax.fori_loop(0, nl, _tb, 0, unroll=4)

    bufs = (buf0, buf1)
    sems = (sem0, sem1)
    for b in range(2):
        @pl.when(b < count)
        def _(b=b):
            j = start + b
            pltpu.async_copy(
                tt_hbm.at[pl.ds(0, d), pl.ds(128 * j, 128)], bufs[b], sems[b])

    @pl.loop(0, base + 2, step=2)
    def _cols(g):
        for b in range(2):
            k = g + b

            @pl.when(k < count)
            def _(k=k, b=b):
                j = start + k
                buf, sem = bufs[b], sems[b]
                pltpu.make_async_copy(
                    tt_hbm.at[pl.ds(0, d), pl.ds(128 * j, 128)], buf, sem
                ).wait()
                _transpose(buf, 128, row_v)
                pltpu.sync_copy(row_v, out_hbm.at[pl.ds(64 * j, 64), :])

                @pl.when(k + 2 < count)
                def _():
                    j2 = start + k + 2
                    pltpu.async_copy(
                        tt_hbm.at[pl.ds(0, d), pl.ds(128 * j2, 128)], buf, sem)

    if rem:
        @pl.when(wid == _NW - 1)
        def _():
            pltpu.sync_copy(tt_hbm.at[pl.ds(0, d), pl.ds(128 * ncol, rem)],
                            buft)
            def _tb(l, _):
                half = (l % 2) * _D
                for c in range(4):
                    rows = 16 * c + lax.iota(jnp.int32, 16)
                    cols = jnp.full((16,), l, jnp.int32)
                    g = plsc.load_gather(buft, [rows, cols])
                    row_v[l // 2, pl.ds(half + 16 * c, 16)] = g
                return 0

            lax.fori_loop(0, rem, _tb, 0, unroll=4)
            pltpu.sync_copy(row_v.at[pl.ds(0, rem // 2)],
                            out_hbm.at[pl.ds(64 * ncol, rem // 2), :])


def _format_table(table_t):
    d, v = table_t.shape
    rem = v - 128 * (v // 128)
    mesh = plsc.VectorSubcoreMesh(core_axis_name="c", subcore_axis_name="s")
    f = pl.kernel(
        _fmt_body,
        out_type=jax.ShapeDtypeStruct((v // 2, 2 * _D), jnp.float32),
        mesh=mesh,
        scratch_types=[
            pltpu.VMEM((d, 128), jnp.float32),
            pltpu.VMEM((d, 128), jnp.float32),
            pltpu.VMEM((d, max(rem, 1)), jnp.float32),
            pltpu.VMEM((64, 2 * _D), jnp.float32),
            pltpu.SemaphoreType.DMA,
            pltpu.SemaphoreType.DMA,
        ],
        compiler_params=pltpu.CompilerParams(needs_layout_passes=False),
    )
    return f(table_t)


def _bag_body(left_hbm, right_hbm, table_hbm, out_l_hbm, out_r_hbm,
              idx_v, bufs, out_v, sems):
    ng = idx_v.shape[0]
    bw = out_v.shape[0]
    wid = lax.axis_index("s") * _NC + lax.axis_index("c")

    for side in range(2):
        names_hbm = left_hbm if side == 0 else right_hbm
        out_hbm = out_l_hbm if side == 0 else out_r_hbm

        pltpu.sync_copy(names_hbm.at[wid], idx_v)
        for b in range(_NBUF):
            pltpu.async_copy(table_hbm.at[idx_v.at[b]], bufs[b], sems[b])

        @pl.loop(0, ng, step=_NBUF)
        def _outer(g):
            for b in range(_NBUF):
                buf, sem = bufs[b], sems[b]
                j = g + b
                pltpu.make_async_copy(table_hbm.at[idx_v.at[j]], buf, sem).wait()
                for bag in range(_PAIR):
                    base = bag * _HIST

                    def _rb(r, acc, base=base, buf=buf):
                        return tuple(
                            acc[c] + buf[base + r, pl.ds(16 * c, 16)]
                            for c in range(4))

                    acc = tuple(buf[base, pl.ds(16 * c, 16)] for c in range(4))
                    acc = lax.fori_loop(1, _HIST, _rb, acc, unroll=7)
                    row = _PAIR * j + bag
                    for c in range(4):
                        out_v[row, pl.ds(16 * c, 16)] = acc[c]

                @pl.when(j + _NBUF < ng)
                def _(buf=buf, sem=sem, j=j):
                    pltpu.async_copy(table_hbm.at[idx_v.at[j + _NBUF]], buf, sem)

        pltpu.sync_copy(out_v, out_hbm.at[pl.ds(wid * bw, bw)])


def _embed_bags(left_idx, right_idx, table):
    nw, ng, g = left_idx.shape
    bw = ng * _PAIR
    b = nw * bw
    mesh = plsc.VectorSubcoreMesh(core_axis_name="c", subcore_axis_name="s")
    f = pl.kernel(
        _bag_body,
        out_type=(jax.ShapeDtypeStruct((b, _D), jnp.float32),
                  jax.ShapeDtypeStruct((b, _D), jnp.float32)),
        mesh=mesh,
        scratch_types=[
            pltpu.VMEM((ng, g), jnp.int32),
            [pltpu.VMEM((g, _D), jnp.float32) for _ in range(_NBUF)],
            pltpu.VMEM((bw, _D), jnp.float32),
            [pltpu.SemaphoreType.DMA for _ in range(_NBUF)],
        ],
        compiler_params=pltpu.CompilerParams(use_tc_tiling_on_sc=False),
    )
    return f(left_idx, right_idx, table)


def _mlp_body(xl_ref, xr_ref, w1_ref, b1_ref, w2_ref, b2_ref, out_ref):
    scale = 1.0 / _HIST
    x = jnp.concatenate((xl_ref[...] * scale, xr_ref[...] * scale), axis=1)
    h = lax.dot_general(x, w1_ref[...], (((1,), (1,)), ((), ())),
                        preferred_element_type=jnp.float32)
    h = jnp.maximum(h + b1_ref[...], 0.0)
    logits = lax.dot_general(h, w2_ref[...], (((1,), (1,)), ((), ())),
                             preferred_element_type=jnp.float32)
    logits = logits + b2_ref[...]
    m = jnp.max(logits, axis=1, keepdims=True)
    e = jnp.exp(logits - m)
    out_ref[...] = e / jnp.sum(e, axis=1, keepdims=True)


def _mlp(xl, xr, w1, b1, w2, b2):
    batch, d = xl.shape
    hidden, two_d = w1.shape
    ncls = w2.shape[0]
    bm = 2048
    grid = (batch // bm,)
    return pl.pallas_call(
        _mlp_body,
        grid=grid,
        in_specs=[
            pl.BlockSpec((bm, d), lambda i: (i, 0)),
            pl.BlockSpec((bm, d), lambda i: (i, 0)),
            pl.BlockSpec((hidden, two_d), lambda i: (0, 0)),
            pl.BlockSpec((1, hidden), lambda i: (0, 0)),
            pl.BlockSpec((ncls, hidden), lambda i: (0, 0)),
            pl.BlockSpec((1, ncls), lambda i: (0, 0)),
        ],
        out_specs=pl.BlockSpec((bm, ncls), lambda i: (i, 0)),
        out_shape=jax.ShapeDtypeStruct((batch, ncls), jnp.float32),
    )(xl, xr, w1, b1, w2, b2)


def kernel(left_names, right_names, emb_table, W1, b1, W2, b2):
    batch, hist = left_names.shape
    ng = batch // (_NW * _PAIR)
    li = left_names.reshape(_NW, ng, _G)
    ri = right_names.reshape(_NW, ng, _G)
    tlin = _format_table(emb_table.T).reshape(emb_table.shape)
    xl, xr = _embed_bags(li, ri, tlin)
    return _mlp(xl, xr, W1, b1.reshape(1, -1), W2, b2.reshape(1, -1))


# speedup vs baseline: 2.2662x; 2.2662x over previous
import functools

import jax
import jax.numpy as jnp
from jax import lax
from jax.experimental import pallas as pl
from jax.experimental.pallas import tpu as pltpu
from jax.experimental.pallas import tpu_sc as plsc

_NC = 2
_NS = 16
_NW = _NC * _NS

_HIST = 50
_D = 64
_PAIR = 2
_G = _PAIR * _HIST
_NBUF = 8


def _bag_body(left_hbm, right_hbm, table_hbm, out_l_hbm, out_r_hbm,
              idx_v, bufs, out_v, sems):
    ng = idx_v.shape[0]
    bw = out_v.shape[0]
    wid = lax.axis_index("s") * _NC + lax.axis_index("c")

    for side in range(2):
        names_hbm = left_hbm if side == 0 else right_hbm
        out_hbm = out_l_hbm if side == 0 else out_r_hbm

        pltpu.sync_copy(names_hbm.at[wid], idx_v)
        for b in range(_NBUF):
            pltpu.async_copy(table_hbm.at[idx_v.at[b]], bufs[b], sems[b])

        @pl.loop(0, ng, step=_NBUF)
        def _outer(g):
            for b in range(_NBUF):
                buf, sem = bufs[b], sems[b]
                j = g + b
                pltpu.make_async_copy(table_hbm.at[idx_v.at[j]], buf, sem).wait()
                for bag in range(_PAIR):
                    base = bag * _HIST

                    def _rb(r, acc, base=base, buf=buf):
                        return tuple(
                            acc[c] + buf[base + r, pl.ds(16 * c, 16)]
                            for c in range(4))

                    acc = tuple(buf[base, pl.ds(16 * c, 16)] for c in range(4))
                    acc = lax.fori_loop(1, _HIST, _rb, acc, unroll=7)
                    row = _PAIR * j + bag
                    for c in range(4):
                        out_v[row, pl.ds(16 * c, 16)] = acc[c]

                @pl.when(j + _NBUF < ng)
                def _(buf=buf, sem=sem, j=j):
                    pltpu.async_copy(table_hbm.at[idx_v.at[j + _NBUF]], buf, sem)

        pltpu.sync_copy(out_v, out_hbm.at[pl.ds(wid * bw, bw)])


def _embed_bags(left_idx, right_idx, table):
    nw, ng, g = left_idx.shape
    bw = ng * _PAIR
    b = nw * bw
    mesh = plsc.VectorSubcoreMesh(core_axis_name="c", subcore_axis_name="s")
    f = pl.kernel(
        _bag_body,
        out_type=(jax.ShapeDtypeStruct((b, _D), jnp.float32),
                  jax.ShapeDtypeStruct((b, _D), jnp.float32)),
        mesh=mesh,
        scratch_types=[
            pltpu.VMEM((ng, g), jnp.int32),
            [pltpu.VMEM((g, _D), jnp.float32) for _ in range(_NBUF)],
            pltpu.VMEM((bw, _D), jnp.float32),
            [pltpu.SemaphoreType.DMA for _ in range(_NBUF)],
        ],
        compiler_params=pltpu.CompilerParams(use_tc_tiling_on_sc=False),
    )
    return f(left_idx, right_idx, table)


def _mlp_body(xl_ref, xr_ref, w1_ref, b1_ref, w2_ref, b2_ref, out_ref):
    scale = 1.0 / _HIST
    x = jnp.concatenate((xl_ref[...] * scale, xr_ref[...] * scale), axis=1)
    h = lax.dot_general(x, w1_ref[...], (((1,), (1,)), ((), ())),
                        preferred_element_type=jnp.float32)
    h = jnp.maximum(h + b1_ref[...], 0.0)
    logits = lax.dot_general(h, w2_ref[...], (((1,), (1,)), ((), ())),
                             preferred_element_type=jnp.float32)
    logits = logits + b2_ref[...]
    m = jnp.max(logits, axis=1, keepdims=True)
    e = jnp.exp(logits - m)
    out_ref[...] = e / jnp.sum(e, axis=1, keepdims=True)


def _mlp(xl, xr, w1, b1, w2, b2):
    batch, d = xl.shape
    hidden, two_d = w1.shape
    ncls = w2.shape[0]
    bm = 4096
    grid = (batch // bm,)
    return pl.pallas_call(
        _mlp_body,
        grid=grid,
        in_specs=[
            pl.BlockSpec((bm, d), lambda i: (i, 0)),
            pl.BlockSpec((bm, d), lambda i: (i, 0)),
            pl.BlockSpec((hidden, two_d), lambda i: (0, 0)),
            pl.BlockSpec((1, hidden), lambda i: (0, 0)),
            pl.BlockSpec((ncls, hidden), lambda i: (0, 0)),
            pl.BlockSpec((1, ncls), lambda i: (0, 0)),
        ],
        out_specs=pl.BlockSpec((bm, ncls), lambda i: (i, 0)),
        out_shape=jax.ShapeDtypeStruct((batch, ncls), jnp.float32),
    )(xl, xr, w1, b1, w2, b2)


def kernel(left_names, right_names, emb_table, W1, b1, W2, b2):
    batch, hist = left_names.shape
    ng = batch // (_NW * _PAIR)
    li = left_names.reshape(_NW, ng, _G)
    ri = right_names.reshape(_NW, ng, _G)
    xl, xr = _embed_bags(li, ri, emb_table)
    return _mlp(xl, xr, W1, b1.reshape(1, -1), W2, b2.reshape(1, -1))
